# baseline (speedup 1.0000x reference)
import functools

import jax
import jax.numpy as jnp
from jax import lax
from jax.experimental import pallas as pl
from jax.experimental.pallas import tpu as pltpu
from jax.experimental.pallas import tpu_sc as plsc

N_TOK = 16384
N_EMB = 8192
HALF = N_EMB // 2
D = 32
B = 256
C = 512
NB = N_TOK // B
NC = N_EMB // C
NCH = NC // 2
COMMIT = 0.25


def _vq_body(xn_ref, en_ref, x_ref, w_ref, inds_ref, dsum_ref):
    xb = x_ref[...]
    xn = xn_ref[...]

    def half_argmin(c0):
        def chunk(k, carry):
            runmin, runidx = carry
            c = c0 + k
            wc = w_ref[c]
            mm = lax.dot_general(xb, wc, (((1,), (1,)), ((), ())),
                                 preferred_element_type=jnp.float32)
            dist = (xn + en_ref[c]) - 2.0 * mm
            minc = jnp.min(dist, axis=1, keepdims=True)
            iota = lax.broadcasted_iota(jnp.int32, (B, C), 1) + k * C
            idxc = jnp.min(jnp.where(dist == minc, iota, N_EMB),
                           axis=1, keepdims=True)
            better = minc < runmin
            return (jnp.where(better, minc, runmin),
                    jnp.where(better, idxc, runidx))
        init = (jnp.full((B, 1), jnp.inf, jnp.float32),
                jnp.zeros((B, 1), jnp.int32))
        return lax.fori_loop(0, NCH, chunk, init)

    vl, il = half_argmin(0)
    vh, ih = half_argmin(NCH)
    vl_b = vl.astype(jnp.bfloat16).astype(jnp.float32)
    hi_wins = vh < vl_b
    inds_ref[...] = jnp.where(hi_wins, ih + HALF, il)

    chosen = jnp.where(hi_wins, vh, vl)
    dsum_ref[...] = jnp.sum(chosen)[None, None, None]


GATHER_W = 128


def _make_sc_gather():
    info = plsc.get_sparse_core_info()
    nw = info.num_cores * info.num_subcores
    bpw = N_TOK // nw
    mesh = plsc.VectorSubcoreMesh(core_axis_name="c", subcore_axis_name="s")

    @functools.partial(
        pl.kernel, mesh=mesh,
        out_type=jax.ShapeDtypeStruct((N_TOK, GATHER_W), jnp.float32),
        scratch_types=[
            pltpu.VMEM((bpw,), jnp.int32),
            pltpu.VMEM((bpw, GATHER_W), jnp.float32),
            pltpu.SemaphoreType.DMA,
        ],
    )
    def gather(table_hbm, idx_hbm, out_hbm, idx_v, rows_v, sem):
        wid = lax.axis_index("s") * info.num_cores + lax.axis_index("c")
        base = wid * bpw
        pltpu.sync_copy(idx_hbm.at[pl.ds(base, bpw)], idx_v)
        pltpu.async_copy(table_hbm.at[idx_v], rows_v, sem).wait()
        pltpu.sync_copy(rows_v, out_hbm.at[pl.ds(base, bpw)])

    return gather


def kernel(x, W):
    xn = jnp.sum(x ** 2, axis=1, keepdims=True)
    en = jnp.sum(W ** 2, axis=1)
    en3 = en.reshape(NC, 1, C)
    w3 = W.reshape(NC, C, D)

    inds2, dsum = pl.pallas_call(
        _vq_body,
        grid=(NB,),
        in_specs=[
            pl.BlockSpec((B, 1), lambda i: (i, 0)),
            pl.BlockSpec((NC, 1, C), lambda i: (0, 0, 0)),
            pl.BlockSpec((B, D), lambda i: (i, 0)),
            pl.BlockSpec((NC, C, D), lambda i: (0, 0, 0)),
        ],
        out_specs=[
            pl.BlockSpec((B, 1), lambda i: (i, 0)),
            pl.BlockSpec((1, 1, 1), lambda i: (i, 0, 0)),
        ],
        out_shape=[
            jax.ShapeDtypeStruct((N_TOK, 1), jnp.int32),
            jax.ShapeDtypeStruct((NB, 1, 1), jnp.float32),
        ],
        compiler_params=pltpu.CompilerParams(
            dimension_semantics=("parallel",)),
    )(xn, en3, x, w3)

    inds = inds2.reshape(N_TOK)
    table = jnp.pad(W, ((0, 0), (0, GATHER_W - D)))
    xq = _make_sc_gather()(table, inds)[:, :D]
    loss = (1.0 + COMMIT) * jnp.sum(dsum) / (N_TOK * D)
    return (xq, loss, inds)


# speedup vs baseline: 1.4112x; 1.4112x over previous
import functools

import jax
import jax.numpy as jnp
from jax import lax
from jax.experimental import pallas as pl
from jax.experimental.pallas import tpu as pltpu
from jax.experimental.pallas import tpu_sc as plsc

N_TOK = 16384
N_EMB = 8192
HALF = N_EMB // 2
D = 32
B = 256
C = 1024
NB = N_TOK // B
NC = N_EMB // C
NCH = NC // 2
COMMIT = 0.25


def _vq_body(xn_ref, en_ref, x_ref, w_ref, inds_ref, dsum_ref):
    xb = x_ref[...]
    xn = xn_ref[...]

    def half_argmin(c0):
        def chunk(k, carry):
            runmin, runidx = carry
            c = c0 + k
            wc = w_ref[c]
            mm = lax.dot_general(xb, wc, (((1,), (1,)), ((), ())),
                                 preferred_element_type=jnp.float32)
            dist = (xn + en_ref[c]) - 2.0 * mm
            minc = jnp.min(dist, axis=1, keepdims=True)
            iota = lax.broadcasted_iota(jnp.int32, (B, C), 1) + k * C
            idxc = jnp.min(jnp.where(dist == minc, iota, N_EMB),
                           axis=1, keepdims=True)
            better = minc < runmin
            return (jnp.where(better, minc, runmin),
                    jnp.where(better, idxc, runidx))
        init = (jnp.full((B, 1), jnp.inf, jnp.float32),
                jnp.zeros((B, 1), jnp.int32))
        return lax.fori_loop(0, NCH, chunk, init)

    vl, il = half_argmin(0)
    vh, ih = half_argmin(NCH)
    vl_b = vl.astype(jnp.bfloat16).astype(jnp.float32)
    hi_wins = vh < vl_b
    inds_ref[...] = jnp.where(hi_wins, ih + HALF, il)

    chosen = jnp.where(hi_wins, vh, vl)
    dsum_ref[...] = jnp.sum(chosen)[None, None, None]


GATHER_W = 128


def _make_sc_gather():
    info = plsc.get_sparse_core_info()
    nw = info.num_cores * info.num_subcores
    bpw = N_TOK // nw
    mesh = plsc.VectorSubcoreMesh(core_axis_name="c", subcore_axis_name="s")

    @functools.partial(
        pl.kernel, mesh=mesh,
        out_type=jax.ShapeDtypeStruct((N_TOK, GATHER_W), jnp.float32),
        scratch_types=[
            pltpu.VMEM((bpw,), jnp.int32),
            pltpu.VMEM((bpw, GATHER_W), jnp.float32),
            pltpu.SemaphoreType.DMA,
        ],
    )
    def gather(table_hbm, idx_hbm, out_hbm, idx_v, rows_v, sem):
        wid = lax.axis_index("s") * info.num_cores + lax.axis_index("c")
        base = wid * bpw
        pltpu.sync_copy(idx_hbm.at[pl.ds(base, bpw)], idx_v)
        pltpu.async_copy(table_hbm.at[idx_v], rows_v, sem).wait()
        pltpu.sync_copy(rows_v, out_hbm.at[pl.ds(base, bpw)])

    return gather


def kernel(x, W):
    xn = jnp.sum(x ** 2, axis=1, keepdims=True)
    en = jnp.sum(W ** 2, axis=1)
    en3 = en.reshape(NC, 1, C)
    w3 = W.reshape(NC, C, D)

    inds2, dsum = pl.pallas_call(
        _vq_body,
        grid=(NB,),
        in_specs=[
            pl.BlockSpec((B, 1), lambda i: (i, 0)),
            pl.BlockSpec((NC, 1, C), lambda i: (0, 0, 0)),
            pl.BlockSpec((B, D), lambda i: (i, 0)),
            pl.BlockSpec((NC, C, D), lambda i: (0, 0, 0)),
        ],
        out_specs=[
            pl.BlockSpec((B, 1), lambda i: (i, 0)),
            pl.BlockSpec((1, 1, 1), lambda i: (i, 0, 0)),
        ],
        out_shape=[
            jax.ShapeDtypeStruct((N_TOK, 1), jnp.int32),
            jax.ShapeDtypeStruct((NB, 1, 1), jnp.float32),
        ],
        compiler_params=pltpu.CompilerParams(
            dimension_semantics=("parallel",)),
    )(xn, en3, x, w3)

    inds = inds2.reshape(N_TOK)
    table = jnp.pad(W, ((0, 0), (0, GATHER_W - D)))
    xq = _make_sc_gather()(table, inds)[:, :D]
    loss = (1.0 + COMMIT) * jnp.sum(dsum) / (N_TOK * D)
    return (xq, loss, inds)


# speedup vs baseline: 1.7668x; 1.2520x over previous
import functools

import jax
import jax.numpy as jnp
from jax import lax
from jax.experimental import pallas as pl
from jax.experimental.pallas import tpu as pltpu
from jax.experimental.pallas import tpu_sc as plsc

N_TOK = 16384
N_EMB = 8192
HALF = N_EMB // 2
D = 32
B = 256
C = 2048
NB = N_TOK // B
NC = N_EMB // C
NCH = NC // 2
COMMIT = 0.25


def _vq_body(xn_ref, en_ref, x_ref, w_ref, inds_ref, dsum_ref):
    xb = x_ref[...]
    xn = xn_ref[...]

    def half_argmin(c0):
        def chunk(k, carry):
            runmin, runidx = carry
            c = c0 + k
            wc = w_ref[c]
            mm = lax.dot_general(xb, wc, (((1,), (1,)), ((), ())),
                                 preferred_element_type=jnp.float32)
            dist = (xn + en_ref[c]) - 2.0 * mm
            minc = jnp.min(dist, axis=1, keepdims=True)
            iota = lax.broadcasted_iota(jnp.int32, (B, C), 1) + k * C
            idxc = jnp.min(jnp.where(dist == minc, iota, N_EMB),
                           axis=1, keepdims=True)
            better = minc < runmin
            return (jnp.where(better, minc, runmin),
                    jnp.where(better, idxc, runidx))
        init = (jnp.full((B, 1), jnp.inf, jnp.float32),
                jnp.zeros((B, 1), jnp.int32))
        return lax.fori_loop(0, NCH, chunk, init)

    vl, il = half_argmin(0)
    vh, ih = half_argmin(NCH)
    vl_b = vl.astype(jnp.bfloat16).astype(jnp.float32)
    hi_wins = vh < vl_b
    inds_ref[...] = jnp.where(hi_wins, ih + HALF, il)

    chosen = jnp.where(hi_wins, vh, vl)
    dsum_ref[...] = jnp.sum(chosen)[None, None, None]


GATHER_W = 128


def _make_sc_gather():
    info = plsc.get_sparse_core_info()
    nw = info.num_cores * info.num_subcores
    bpw = N_TOK // nw
    mesh = plsc.VectorSubcoreMesh(core_axis_name="c", subcore_axis_name="s")

    @functools.partial(
        pl.kernel, mesh=mesh,
        out_type=jax.ShapeDtypeStruct((N_TOK, GATHER_W), jnp.float32),
        scratch_types=[
            pltpu.VMEM((bpw,), jnp.int32),
            pltpu.VMEM((bpw, GATHER_W), jnp.float32),
            pltpu.SemaphoreType.DMA,
        ],
    )
    def gather(table_hbm, idx_hbm, out_hbm, idx_v, rows_v, sem):
        wid = lax.axis_index("s") * info.num_cores + lax.axis_index("c")
        base = wid * bpw
        pltpu.sync_copy(idx_hbm.at[pl.ds(base, bpw)], idx_v)
        pltpu.async_copy(table_hbm.at[idx_v], rows_v, sem).wait()
        pltpu.sync_copy(rows_v, out_hbm.at[pl.ds(base, bpw)])

    return gather


def kernel(x, W):
    xn = jnp.sum(x ** 2, axis=1, keepdims=True)
    en = jnp.sum(W ** 2, axis=1)
    en3 = en.reshape(NC, 1, C)
    w3 = W.reshape(NC, C, D)

    inds2, dsum = pl.pallas_call(
        _vq_body,
        grid=(NB,),
        in_specs=[
            pl.BlockSpec((B, 1), lambda i: (i, 0)),
            pl.BlockSpec((NC, 1, C), lambda i: (0, 0, 0)),
            pl.BlockSpec((B, D), lambda i: (i, 0)),
            pl.BlockSpec((NC, C, D), lambda i: (0, 0, 0)),
        ],
        out_specs=[
            pl.BlockSpec((B, 1), lambda i: (i, 0)),
            pl.BlockSpec((1, 1, 1), lambda i: (i, 0, 0)),
        ],
        out_shape=[
            jax.ShapeDtypeStruct((N_TOK, 1), jnp.int32),
            jax.ShapeDtypeStruct((NB, 1, 1), jnp.float32),
        ],
        compiler_params=pltpu.CompilerParams(
            dimension_semantics=("parallel",)),
    )(xn, en3, x, w3)

    inds = inds2.reshape(N_TOK)
    table = jnp.pad(W, ((0, 0), (0, GATHER_W - D)))
    xq = _make_sc_gather()(table, inds)[:, :D]
    loss = (1.0 + COMMIT) * jnp.sum(dsum) / (N_TOK * D)
    return (xq, loss, inds)


# speedup vs baseline: 2.4152x; 1.3670x over previous
import functools

import jax
import jax.numpy as jnp
from jax import lax
from jax.experimental import pallas as pl
from jax.experimental.pallas import tpu as pltpu
from jax.experimental.pallas import tpu_sc as plsc

N_TOK = 16384
N_EMB = 8192
HALF = N_EMB // 2
D = 32
B = 256
C = 4096
NB = N_TOK // B
NC = N_EMB // C
NCH = NC // 2
COMMIT = 0.25


def _vq_body(xn_ref, en_ref, x_ref, w_ref, inds_ref, dsum_ref):
    xb = x_ref[...]
    xn = xn_ref[...]

    def half_argmin(c0):
        def chunk(k, carry):
            runmin, runidx = carry
            c = c0 + k
            wc = w_ref[c]
            mm = lax.dot_general(xb, wc, (((1,), (1,)), ((), ())),
                                 preferred_element_type=jnp.float32)
            dist = (xn + en_ref[c]) - 2.0 * mm
            minc = jnp.min(dist, axis=1, keepdims=True)
            iota = lax.broadcasted_iota(jnp.int32, (B, C), 1) + k * C
            idxc = jnp.min(jnp.where(dist == minc, iota, N_EMB),
                           axis=1, keepdims=True)
            better = minc < runmin
            return (jnp.where(better, minc, runmin),
                    jnp.where(better, idxc, runidx))
        init = (jnp.full((B, 1), jnp.inf, jnp.float32),
                jnp.zeros((B, 1), jnp.int32))
        return lax.fori_loop(0, NCH, chunk, init)

    vl, il = half_argmin(0)
    vh, ih = half_argmin(NCH)
    vl_b = vl.astype(jnp.bfloat16).astype(jnp.float32)
    hi_wins = vh < vl_b
    inds_ref[...] = jnp.where(hi_wins, ih + HALF, il)

    chosen = jnp.where(hi_wins, vh, vl)
    dsum_ref[...] = jnp.sum(chosen)[None, None, None]


GATHER_W = 128


def _make_sc_gather():
    info = plsc.get_sparse_core_info()
    nw = info.num_cores * info.num_subcores
    bpw = N_TOK // nw
    mesh = plsc.VectorSubcoreMesh(core_axis_name="c", subcore_axis_name="s")

    @functools.partial(
        pl.kernel, mesh=mesh,
        out_type=jax.ShapeDtypeStruct((N_TOK, GATHER_W), jnp.float32),
        scratch_types=[
            pltpu.VMEM((bpw,), jnp.int32),
            pltpu.VMEM((bpw, GATHER_W), jnp.float32),
            pltpu.SemaphoreType.DMA,
        ],
    )
    def gather(table_hbm, idx_hbm, out_hbm, idx_v, rows_v, sem):
        wid = lax.axis_index("s") * info.num_cores + lax.axis_index("c")
        base = wid * bpw
        pltpu.sync_copy(idx_hbm.at[pl.ds(base, bpw)], idx_v)
        pltpu.async_copy(table_hbm.at[idx_v], rows_v, sem).wait()
        pltpu.sync_copy(rows_v, out_hbm.at[pl.ds(base, bpw)])

    return gather


def kernel(x, W):
    xn = jnp.sum(x ** 2, axis=1, keepdims=True)
    en = jnp.sum(W ** 2, axis=1)
    en3 = en.reshape(NC, 1, C)
    w3 = W.reshape(NC, C, D)

    inds2, dsum = pl.pallas_call(
        _vq_body,
        grid=(NB,),
        in_specs=[
            pl.BlockSpec((B, 1), lambda i: (i, 0)),
            pl.BlockSpec((NC, 1, C), lambda i: (0, 0, 0)),
            pl.BlockSpec((B, D), lambda i: (i, 0)),
            pl.BlockSpec((NC, C, D), lambda i: (0, 0, 0)),
        ],
        out_specs=[
            pl.BlockSpec((B, 1), lambda i: (i, 0)),
            pl.BlockSpec((1, 1, 1), lambda i: (i, 0, 0)),
        ],
        out_shape=[
            jax.ShapeDtypeStruct((N_TOK, 1), jnp.int32),
            jax.ShapeDtypeStruct((NB, 1, 1), jnp.float32),
        ],
        compiler_params=pltpu.CompilerParams(
            dimension_semantics=("parallel",)),
    )(xn, en3, x, w3)

    inds = inds2.reshape(N_TOK)
    table = jnp.pad(W, ((0, 0), (0, GATHER_W - D)))
    xq = _make_sc_gather()(table, inds)[:, :D]
    loss = (1.0 + COMMIT) * jnp.sum(dsum) / (N_TOK * D)
    return (xq, loss, inds)


# speedup vs baseline: 2.4998x; 1.0350x over previous
import functools

import jax
import jax.numpy as jnp
from jax import lax
from jax.experimental import pallas as pl
from jax.experimental.pallas import tpu as pltpu
from jax.experimental.pallas import tpu_sc as plsc

N_TOK = 16384
N_EMB = 8192
HALF = N_EMB // 2
D = 32
B = 512
C = 4096
NB = N_TOK // B
NC = N_EMB // C
NCH = NC // 2
COMMIT = 0.25


def _vq_body(xn_ref, en_ref, x_ref, w_ref, inds_ref, dsum_ref):
    xb = x_ref[...]
    xn = xn_ref[...]

    def half_argmin(c0):
        def chunk(k, carry):
            runmin, runidx = carry
            c = c0 + k
            wc = w_ref[c]
            mm = lax.dot_general(xb, wc, (((1,), (1,)), ((), ())),
                                 preferred_element_type=jnp.float32)
            dist = (xn + en_ref[c]) - 2.0 * mm
            minc = jnp.min(dist, axis=1, keepdims=True)
            iota = lax.broadcasted_iota(jnp.int32, (B, C), 1) + k * C
            idxc = jnp.min(jnp.where(dist == minc, iota, N_EMB),
                           axis=1, keepdims=True)
            better = minc < runmin
            return (jnp.where(better, minc, runmin),
                    jnp.where(better, idxc, runidx))
        init = (jnp.full((B, 1), jnp.inf, jnp.float32),
                jnp.zeros((B, 1), jnp.int32))
        return lax.fori_loop(0, NCH, chunk, init)

    vl, il = half_argmin(0)
    vh, ih = half_argmin(NCH)
    vl_b = vl.astype(jnp.bfloat16).astype(jnp.float32)
    hi_wins = vh < vl_b
    inds_ref[...] = jnp.where(hi_wins, ih + HALF, il)

    chosen = jnp.where(hi_wins, vh, vl)
    dsum_ref[...] = jnp.sum(chosen)[None, None, None]


GATHER_W = 128


def _make_sc_gather():
    info = plsc.get_sparse_core_info()
    nw = info.num_cores * info.num_subcores
    bpw = N_TOK // nw
    mesh = plsc.VectorSubcoreMesh(core_axis_name="c", subcore_axis_name="s")

    @functools.partial(
        pl.kernel, mesh=mesh,
        out_type=jax.ShapeDtypeStruct((N_TOK, GATHER_W), jnp.float32),
        scratch_types=[
            pltpu.VMEM((bpw,), jnp.int32),
            pltpu.VMEM((bpw, GATHER_W), jnp.float32),
            pltpu.SemaphoreType.DMA,
        ],
    )
    def gather(table_hbm, idx_hbm, out_hbm, idx_v, rows_v, sem):
        wid = lax.axis_index("s") * info.num_cores + lax.axis_index("c")
        base = wid * bpw
        pltpu.sync_copy(idx_hbm.at[pl.ds(base, bpw)], idx_v)
        pltpu.async_copy(table_hbm.at[idx_v], rows_v, sem).wait()
        pltpu.sync_copy(rows_v, out_hbm.at[pl.ds(base, bpw)])

    return gather


def kernel(x, W):
    xn = jnp.sum(x ** 2, axis=1, keepdims=True)
    en = jnp.sum(W ** 2, axis=1)
    en3 = en.reshape(NC, 1, C)
    w3 = W.reshape(NC, C, D)

    inds2, dsum = pl.pallas_call(
        _vq_body,
        grid=(NB,),
        in_specs=[
            pl.BlockSpec((B, 1), lambda i: (i, 0)),
            pl.BlockSpec((NC, 1, C), lambda i: (0, 0, 0)),
            pl.BlockSpec((B, D), lambda i: (i, 0)),
            pl.BlockSpec((NC, C, D), lambda i: (0, 0, 0)),
        ],
        out_specs=[
            pl.BlockSpec((B, 1), lambda i: (i, 0)),
            pl.BlockSpec((1, 1, 1), lambda i: (i, 0, 0)),
        ],
        out_shape=[
            jax.ShapeDtypeStruct((N_TOK, 1), jnp.int32),
            jax.ShapeDtypeStruct((NB, 1, 1), jnp.float32),
        ],
        compiler_params=pltpu.CompilerParams(
            dimension_semantics=("parallel",)),
    )(xn, en3, x, w3)

    inds = inds2.reshape(N_TOK)
    table = jnp.pad(W, ((0, 0), (0, GATHER_W - D)))
    xq = _make_sc_gather()(table, inds)[:, :D]
    loss = (1.0 + COMMIT) * jnp.sum(dsum) / (N_TOK * D)
    return (xq, loss, inds)
